# baseline (speedup 1.0000x reference)
import functools

import jax
import jax.numpy as jnp
from jax import lax
from jax.experimental import pallas as pl
from jax.experimental.pallas import tpu as pltpu
from jax.experimental.pallas import tpu_sc as plsc

_F32 = jnp.float32
_SELU_SCALE = 1.0507009873554805
_SELU_ALPHA = 1.6732632423543772


def _selu(x):
    neg = _SELU_ALPHA * (jnp.exp(jnp.minimum(x, 0.0)) - 1.0)
    return jnp.where(x > 0, _SELU_SCALE * x, _SELU_SCALE * neg)


def _selu_u(x):
    return jnp.where(x > 0, x,
                     _SELU_ALPHA * jnp.exp(jnp.minimum(x, 0.0)) - _SELU_ALPHA)


def _pad2(w, r, c):
    w = jnp.asarray(w, _F32)
    return jnp.zeros((r, c), _F32).at[: w.shape[0], : w.shape[1]].set(w)


def _row(b, c):
    b = jnp.asarray(b, _F32).reshape(1, -1)
    return _pad2(b, 1, c)


def _bdg(w, gi, go):
    return jnp.kron(jnp.eye(8, dtype=_F32), _pad2(w, gi, go))


def _tileg(b, g):
    return jnp.tile(_row(b, g), (1, 8))


def kernel(nodes, edges, senders, receivers, params):
    N, E = nodes.shape[0], edges.shape[0]
    NBK = 2048
    EBK = 1024
    npad = -(-N // NBK) * NBK
    epad = -(-E // 32768) * 32768
    e8 = epad // 8

    senders = jnp.asarray(senders, jnp.int32)
    receivers = jnp.asarray(receivers, jnp.int32)
    pad_e = epad - E

    nblk = epad // 1024

    def _perm(x):
        return x.reshape(nblk, 8, 128).transpose(0, 2, 1).reshape(epad)

    pad_gather = jnp.arange(pad_e, dtype=jnp.int32) % N
    pad_scatter = N + (jnp.arange(pad_e, dtype=jnp.int32) % 8)
    ri_g = _perm(jnp.concatenate([receivers, pad_gather]))
    si_g = _perm(jnp.concatenate([senders, pad_gather]))
    ri_sc = _perm(jnp.concatenate([receivers, pad_scatter])).reshape(
        epad // 128, 128)

    nodes_p8 = _pad2(nodes, npad, 8)
    edt = jnp.pad(edges.astype(_F32).T, ((0, 0), (0, pad_e)))
    ecols = [edt[j].reshape(epad // 128, 128) for j in range(3)]

    pn, ue, pe, un, pr = (params["permutate_nodes"], params["update_edges"],
                          params["permutate_edges"], params["update_nodes"],
                          params["predict"])
    w1r = _pad2(pn[0][0][:3], 8, 16)
    w1s = _pad2(pn[0][0][3:], 8, 16)
    b1t = _tileg(pn[0][1], 16)
    _S = _SELU_SCALE
    w2bd, b2t = _bdg(_S * pn[1][0], 16, 32), _tileg(pn[1][1], 32)
    w3bd, b3t = _bdg(_S * pn[2][0], 32, 16), _tileg(pn[2][1], 16)
    uw1e = _bdg(ue[0][0][:3], 16, 32)
    qmats = [jnp.kron(jnp.eye(8, dtype=_F32), _row(ue[0][0][j], 32))
             for j in range(3)]
    uw1n = _bdg(ue[0][0][3:], 16, 32)
    ub1t = _tileg(ue[0][1], 32)
    uw2bd, ub2t = _bdg(_S * ue[1][0], 32, 16), _tileg(ue[1][1], 16)
    uw3bd, ub3t = _bdg(_S * ue[2][0], 16, 16), _tileg(ue[2][1], 16)
    pw1, pb1 = _pad2(pe[0][0], 8, 16), _row(pe[0][1], 16)
    pw2, pb2 = _pad2(_S * pe[1][0], 16, 32), _row(pe[1][1], 32)
    pw3, pb3 = _pad2(_S * pe[2][0], 32, 16), _row(pe[2][1], 16)
    nw1n = _pad2(un[0][0][:3], 8, 32)
    nw1e = _pad2(un[0][0][3:], 16, 32)
    nb1 = _row(un[0][1], 32)
    nw2, nb2 = _pad2(_S * un[1][0], 32, 16), _row(un[1][1], 16)
    nw3, nb3 = _pad2(_S * un[2][0], 16, 8), _row(un[2][1], 8)
    qw1, qb1 = _pad2(pr[0][0], 24, 16), _row(pr[0][1], 16)
    qw2, qb2 = _pad2(_S * pr[1][0], 16, 16), _row(pr[1][1], 16)
    qw3, qb3 = _pad2(_S * pr[2][0], 16, 16), _row(pr[2][1], 16)

    def _tables_k(x_ref, wr_ref, ws_ref, tr_ref, ts_ref):
        x = x_ref[...]
        tr_ref[...] = jnp.dot(x, wr_ref[...], preferred_element_type=_F32)
        ts_ref[...] = jnp.dot(x, ws_ref[...], preferred_element_type=_F32)

    tr, ts = pl.pallas_call(
        _tables_k,
        grid=(npad // NBK,),
        in_specs=[
            pl.BlockSpec((NBK, 8), lambda i: (i, 0)),
            pl.BlockSpec((8, 16), lambda i: (0, 0)),
            pl.BlockSpec((8, 16), lambda i: (0, 0)),
        ],
        out_specs=[pl.BlockSpec((NBK, 16), lambda i: (i, 0))] * 2,
        out_shape=[jax.ShapeDtypeStruct((npad, 16), _F32)] * 2,
    )(nodes_p8, w1r, w1s)

    chg = 1024
    per_tile = epad // 16
    nch = per_tile // chg
    mesh = plsc.VectorSubcoreMesh(core_axis_name="c", subcore_axis_name="s")

    @functools.partial(
        pl.kernel,
        out_type=(jax.ShapeDtypeStruct((epad, 16), _F32),
                  jax.ShapeDtypeStruct((epad, 16), _F32)),
        mesh=mesh,
        scratch_types=[
            pltpu.VMEM((chg,), jnp.int32),
            pltpu.VMEM((chg, 16), _F32),
            pltpu.SemaphoreType.DMA,
        ],
        compiler_params=pltpu.CompilerParams(use_tc_tiling_on_sc=False),
    )
    def _gather_k(tr_hbm, ts_hbm, ri_hbm, si_hbm, outr, outs, idx_v, rows_v, sem):
        c = lax.axis_index("c")
        s = lax.axis_index("s")

        def run(tab, idxs, out):
            def body(i):
                base = s * per_tile + i * chg
                pltpu.sync_copy(idxs.at[pl.ds(base, chg)], idx_v)
                pltpu.async_copy(tab.at[idx_v], rows_v, sem).wait()
                pltpu.sync_copy(rows_v, out.at[pl.ds(base, chg)])
            pl.loop(0, nch)(body)

        @pl.when(c == 0)
        def _():
            run(tr_hbm, ri_hbm, outr)

        @pl.when(c == 1)
        def _():
            run(ts_hbm, si_hbm, outs)

    trg, tsg = _gather_k(tr, ts, ri_g, si_g)
    trg = trg.reshape(e8, 128)
    tsg = tsg.reshape(e8, 128)

    BR = EBK // 8

    def _edge_k(trg_r, tsg_r, e0_r, e1_r, e2_r, q0_r, q1_r, q2_r, b1_r, w2_r,
                b2_r, w3_r, b3_r, w1n_r, w1e_r, ub1_r, uw2_r, ub2_r, uw3_r,
                ub3_r, out_r):
        h = _selu_u(trg_r[...] + tsg_r[...] + b1_r[...])
        h = _selu_u(jnp.dot(h, w2_r[...], preferred_element_type=_F32) + b2_r[...])
        hn = jnp.dot(h, w3_r[...], preferred_element_type=_F32) + b3_r[...]
        c = jnp.dot(hn, w1n_r[...], preferred_element_type=_F32) + ub1_r[...]
        dn = (((0,), (0,)), ((), ()))
        edc = (lax.dot_general(e0_r[...], q0_r[...], dn, preferred_element_type=_F32)
               + lax.dot_general(e1_r[...], q1_r[...], dn, preferred_element_type=_F32)
               + lax.dot_general(e2_r[...], q2_r[...], dn, preferred_element_type=_F32))
        t = _selu_u(edc + c)
        t = _selu_u(jnp.dot(t, uw2_r[...], preferred_element_type=_F32) + ub2_r[...])
        he = jnp.dot(t, uw3_r[...], preferred_element_type=_F32) + ub3_r[...]
        for _ in range(2):
            t = _selu_u(jnp.dot(he, w1e_r[...], preferred_element_type=_F32) + c)
            t = _selu_u(jnp.dot(t, uw2_r[...], preferred_element_type=_F32) + ub2_r[...])
            he = jnp.dot(t, uw3_r[...], preferred_element_type=_F32) + ub3_r[...]
        out_r[...] = he

    _w = lambda shape: pl.BlockSpec(shape, lambda i: (0, 0))
    he_pk = pl.pallas_call(
        _edge_k,
        grid=(e8 // BR,),
        in_specs=[
            pl.BlockSpec((BR, 128), lambda i: (i, 0)),
            pl.BlockSpec((BR, 128), lambda i: (i, 0)),
            pl.BlockSpec((8, 128), lambda i: (i, 0)),
            pl.BlockSpec((8, 128), lambda i: (i, 0)),
            pl.BlockSpec((8, 128), lambda i: (i, 0)),
            _w((8, 256)), _w((8, 256)), _w((8, 256)),
            _w((1, 128)), _w((128, 256)), _w((1, 256)), _w((256, 128)),
            _w((1, 128)), _w((128, 256)), _w((128, 256)), _w((1, 256)),
            _w((256, 128)), _w((1, 128)), _w((128, 128)), _w((1, 128)),
        ],
        out_specs=pl.BlockSpec((BR, 128), lambda i: (i, 0)),
        out_shape=jax.ShapeDtypeStruct((e8, 128), _F32),
    )(trg, tsg, ecols[0], ecols[1], ecols[2], qmats[0], qmats[1], qmats[2],
      b1t, w2bd, b2t, w3bd, b3t, uw1n, uw1e, ub1t, uw2bd, ub2t, uw3bd, ub3t)
    he = he_pk.reshape(epad, 16)

    chs = 1024
    sub = chs // 128
    per_w = epad // 32
    nchs = per_w // chs
    rows_per_tile = npad // 16
    zeros_hbm = jnp.zeros((npad, 8), _F32)
    mesh2 = plsc.VectorSubcoreMesh(core_axis_name="c", subcore_axis_name="s")

    @functools.partial(
        pl.kernel,
        out_type=(jax.ShapeDtypeStruct((npad, 8), _F32),
                  jax.ShapeDtypeStruct((npad, 8), _F32)),
        mesh=mesh2,
        scratch_types=[
            pltpu.VMEM((sub, 128), jnp.int32),
            pltpu.VMEM((chs, 8), _F32),
            pltpu.VMEM_SHARED((npad, 8), _F32),
            pltpu.SemaphoreType.DMA,
        ],
        compiler_params=pltpu.CompilerParams(use_tc_tiling_on_sc=False),
    )
    def _scatter_k(he_hbm, ri_hbm, z_hbm, out0, out1, idx_v, val_v, acc, sem):
        c = lax.axis_index("c")
        s = lax.axis_index("s")
        wid = s * 2 + c
        tile_rows = pl.ds(s * rows_per_tile, rows_per_tile)
        pltpu.sync_copy(z_hbm.at[tile_rows], acc.at[tile_rows])
        plsc.subcore_barrier()

        def body(i):
            ebase = wid * per_w + i * chs
            pltpu.sync_copy(ri_hbm.at[pl.ds(ebase // 128, sub)], idx_v)
            pltpu.sync_copy(he_hbm.at[pl.ds(ebase, chs), pl.ds(0, 8)], val_v)
            for j in range(sub):
                pltpu.sync_copy(val_v.at[pl.ds(j * 128, 128)],
                                acc.at[idx_v.at[j]], add=True)

        pl.loop(0, nchs)(body)
        plsc.subcore_barrier()

        @pl.when(c == 0)
        def _():
            pltpu.sync_copy(acc.at[tile_rows], out0.at[tile_rows])

        @pl.when(c == 1)
        def _():
            pltpu.sync_copy(acc.at[tile_rows], out1.at[tile_rows])

    p0, p1 = _scatter_k(he, ri_sc, zeros_hbm)

    def _node_k(p0_r, p1_r, nd_r, pw1_r, pb1_r, pw2_r, pb2_r, pw3_r, pb3_r,
                nw1n_r, nw1e_r, nb1_r, nw2_r, nb2_r, nw3_r, nb3_r, out_r):
        i = pl.program_id(0)
        seg = p0_r[...] + p1_r[...]
        h = _selu_u(jnp.dot(seg, pw1_r[...], preferred_element_type=_F32) + pb1_r[...])
        h = _selu_u(jnp.dot(h, pw2_r[...], preferred_element_type=_F32) + pb2_r[...])
        h2e = jnp.dot(h, pw3_r[...], preferred_element_type=_F32) + pb3_r[...]
        c2 = jnp.dot(h2e, nw1e_r[...], preferred_element_type=_F32) + nb1_r[...]
        hn = nd_r[...]
        for _ in range(3):
            t = _selu_u(jnp.dot(hn, nw1n_r[...], preferred_element_type=_F32) + c2)
            t = _selu_u(jnp.dot(t, nw2_r[...], preferred_element_type=_F32) + nb2_r[...])
            hn = jnp.dot(t, nw3_r[...], preferred_element_type=_F32) + nb3_r[...]
        g = jnp.concatenate([hn[:, :3], h2e, jnp.zeros((NBK, 5), _F32)], axis=1)
        rows = i * NBK + lax.broadcasted_iota(jnp.int32, (NBK, 1), 0)
        g = jnp.where(rows < N, g, 0.0)
        gs = jnp.sum(g, axis=0, keepdims=True)
        contrib = jnp.concatenate([gs, jnp.zeros((7, 24), _F32)], axis=0)

        @pl.when(i == 0)
        def _():
            out_r[...] = jnp.zeros((8, 24), _F32)

        out_r[...] += contrib

    gsum = pl.pallas_call(
        _node_k,
        grid=(npad // NBK,),
        in_specs=[
            pl.BlockSpec((NBK, 8), lambda i: (i, 0)),
            pl.BlockSpec((NBK, 8), lambda i: (i, 0)),
            pl.BlockSpec((NBK, 8), lambda i: (i, 0)),
            _w((8, 16)), _w((1, 16)), _w((16, 32)), _w((1, 32)), _w((32, 16)),
            _w((1, 16)), _w((8, 32)), _w((16, 32)), _w((1, 32)), _w((32, 16)),
            _w((1, 16)), _w((16, 8)), _w((1, 8)),
        ],
        out_specs=pl.BlockSpec((8, 24), lambda i: (0, 0)),
        out_shape=jax.ShapeDtypeStruct((8, 24), _F32),
    )(p0, p1, nodes_p8, pw1, pb1, pw2, pb2, pw3, pb3, nw1n, nw1e, nb1,
      nw2, nb2, nw3, nb3)

    def _pred_k(x_r, w1_r, b1_r, w2_r, b2_r, w3_r, b3_r, out_r):
        h = _selu_u(jnp.dot(x_r[...], w1_r[...], preferred_element_type=_F32) + b1_r[...])
        h = _selu_u(jnp.dot(h, w2_r[...], preferred_element_type=_F32) + b2_r[...])
        out_r[...] = jnp.dot(h, w3_r[...], preferred_element_type=_F32) + b3_r[...]

    out = pl.pallas_call(
        _pred_k,
        out_shape=jax.ShapeDtypeStruct((8, 16), _F32),
    )(gsum, qw1, qb1, qw2, qb2, qw3, qb3)

    return out[0:1, 0:9]


# speedup vs baseline: 1.0339x; 1.0339x over previous
import functools

import jax
import jax.numpy as jnp
from jax import lax
from jax.experimental import pallas as pl
from jax.experimental.pallas import tpu as pltpu
from jax.experimental.pallas import tpu_sc as plsc

_F32 = jnp.float32
_SELU_SCALE = 1.0507009873554805
_SELU_ALPHA = 1.6732632423543772


def _selu(x):
    neg = _SELU_ALPHA * (jnp.exp(x) - 1.0)
    return jnp.where(x > 0, _SELU_SCALE * x, _SELU_SCALE * neg)


def _pad2(w, r, c):
    w = jnp.asarray(w, _F32)
    return jnp.zeros((r, c), _F32).at[: w.shape[0], : w.shape[1]].set(w)


def _row(b, c):
    b = jnp.asarray(b, _F32).reshape(1, -1)
    return _pad2(b, 1, c)


def _bdg(w, gi, go):
    return jnp.kron(jnp.eye(8, dtype=_F32), _pad2(w, gi, go))


def _tileg(b, g):
    return jnp.tile(_row(b, g), (1, 8))


def kernel(nodes, edges, senders, receivers, params):
    N, E = nodes.shape[0], edges.shape[0]
    NBK = 2048
    npad = -(-N // NBK) * NBK
    epad = -(-E // 65536) * 65536
    eh = epad // 2
    e8h = eh // 8

    senders = jnp.asarray(senders, jnp.int32)
    receivers = jnp.asarray(receivers, jnp.int32)
    pad_e = epad - E

    nblk = epad // 1024

    def _perm(x):
        return x.reshape(nblk, 8, 128).transpose(0, 2, 1).reshape(epad)

    pad_gather = jnp.arange(pad_e, dtype=jnp.int32) % N
    pad_scatter = N + (jnp.arange(pad_e, dtype=jnp.int32) % 8)
    ri_g = _perm(jnp.concatenate([receivers, pad_gather]))
    si_g = _perm(jnp.concatenate([senders, pad_gather]))
    ri_sc = _perm(jnp.concatenate([receivers, pad_scatter])).reshape(
        epad // 128, 128)

    nodes_p8 = _pad2(nodes, npad, 8)
    edt = jnp.pad(edges.astype(_F32).T, ((0, 0), (0, pad_e)))
    ecols = [[edt[j, h * eh:(h + 1) * eh].reshape(eh // 128, 128)
              for j in range(3)] for h in range(2)]

    pn, ue, pe, un, pr = (params["permutate_nodes"], params["update_edges"],
                          params["permutate_edges"], params["update_nodes"],
                          params["predict"])
    w1r = _pad2(pn[0][0][:3], 8, 16)
    w1s = _pad2(pn[0][0][3:], 8, 16)
    b1t = _tileg(pn[0][1], 16)
    w2bd, b2t = _bdg(pn[1][0], 16, 32), _tileg(pn[1][1], 32)
    w3bd, b3t = _bdg(pn[2][0], 32, 16), _tileg(pn[2][1], 16)
    uw1e = _bdg(ue[0][0][:3], 16, 32)
    qmats = [jnp.kron(jnp.eye(8, dtype=_F32), _row(ue[0][0][j], 32))
             for j in range(3)]
    uw1n = _bdg(ue[0][0][3:], 16, 32)
    ub1t = _tileg(ue[0][1], 32)
    uw2bd, ub2t = _bdg(ue[1][0], 32, 32), _tileg(ue[1][1], 32)
    uw3bd, ub3t = _bdg(ue[2][0], 32, 16), _tileg(ue[2][1], 16)
    pw1, pb1 = _pad2(pe[0][0], 8, 16), _row(pe[0][1], 16)
    pw2, pb2 = _pad2(pe[1][0], 16, 32), _row(pe[1][1], 32)
    pw3, pb3 = _pad2(pe[2][0], 32, 16), _row(pe[2][1], 16)
    nw1n = _pad2(un[0][0][:3], 8, 32)
    nw1e = _pad2(un[0][0][3:], 16, 32)
    nb1 = _row(un[0][1], 32)
    nw2, nb2 = _pad2(un[1][0], 32, 16), _row(un[1][1], 16)
    nw3, nb3 = _pad2(un[2][0], 16, 8), _row(un[2][1], 8)
    qw1, qb1 = _pad2(pr[0][0], 24, 16), _row(pr[0][1], 16)
    qw2, qb2 = _pad2(pr[1][0], 16, 16), _row(pr[1][1], 16)
    qw3, qb3 = _pad2(pr[2][0], 16, 16), _row(pr[2][1], 16)

    def _tables_k(x_ref, wr_ref, ws_ref, tr_ref, ts_ref):
        x = x_ref[...]
        tr_ref[...] = jnp.dot(x, wr_ref[...], preferred_element_type=_F32)
        ts_ref[...] = jnp.dot(x, ws_ref[...], preferred_element_type=_F32)

    tr, ts = pl.pallas_call(
        _tables_k,
        grid=(npad // NBK,),
        in_specs=[
            pl.BlockSpec((NBK, 8), lambda i: (i, 0)),
            pl.BlockSpec((8, 16), lambda i: (0, 0)),
            pl.BlockSpec((8, 16), lambda i: (0, 0)),
        ],
        out_specs=[pl.BlockSpec((NBK, 16), lambda i: (i, 0))] * 2,
        out_shape=[jax.ShapeDtypeStruct((npad, 16), _F32)] * 2,
    )(nodes_p8, w1r, w1s)

    chg = 1024
    per_tile = eh // 16
    nch = per_tile // chg
    mesh = plsc.VectorSubcoreMesh(core_axis_name="c", subcore_axis_name="s")

    @functools.partial(
        pl.kernel,
        out_type=(jax.ShapeDtypeStruct((eh, 16), _F32),
                  jax.ShapeDtypeStruct((eh, 16), _F32)),
        mesh=mesh,
        scratch_types=[
            pltpu.VMEM((chg,), jnp.int32),
            pltpu.VMEM((chg, 16), _F32),
            pltpu.SemaphoreType.DMA,
        ],
        compiler_params=pltpu.CompilerParams(use_tc_tiling_on_sc=False),
    )
    def _gather_k(tr_hbm, ts_hbm, ri_hbm, si_hbm, outr, outs, idx_v, rows_v, sem):
        c = lax.axis_index("c")
        s = lax.axis_index("s")

        def run(tab, idxs, out):
            def body(i):
                base = s * per_tile + i * chg
                pltpu.sync_copy(idxs.at[pl.ds(base, chg)], idx_v)
                pltpu.async_copy(tab.at[idx_v], rows_v, sem).wait()
                pltpu.sync_copy(rows_v, out.at[pl.ds(base, chg)])
            pl.loop(0, nch)(body)

        @pl.when(c == 0)
        def _():
            run(tr_hbm, ri_hbm, outr)

        @pl.when(c == 1)
        def _():
            run(ts_hbm, si_hbm, outs)

    BR = 128

    def _edge_k(trg_r, tsg_r, e0_r, e1_r, e2_r, q0_r, q1_r, q2_r, b1_r, w2_r,
                b2_r, w3_r, b3_r, w1n_r, w1e_r, ub1_r, uw2_r, ub2_r, uw3_r,
                ub3_r, out_r):
        h = _selu(trg_r[...] + tsg_r[...] + b1_r[...])
        h = _selu(jnp.dot(h, w2_r[...], preferred_element_type=_F32) + b2_r[...])
        hn = jnp.dot(h, w3_r[...], preferred_element_type=_F32) + b3_r[...]
        c = jnp.dot(hn, w1n_r[...], preferred_element_type=_F32) + ub1_r[...]
        dn = (((0,), (0,)), ((), ()))
        edc = (lax.dot_general(e0_r[...], q0_r[...], dn, preferred_element_type=_F32)
               + lax.dot_general(e1_r[...], q1_r[...], dn, preferred_element_type=_F32)
               + lax.dot_general(e2_r[...], q2_r[...], dn, preferred_element_type=_F32))
        t = _selu(edc + c)
        t = _selu(jnp.dot(t, uw2_r[...], preferred_element_type=_F32) + ub2_r[...])
        he = jnp.dot(t, uw3_r[...], preferred_element_type=_F32) + ub3_r[...]
        for _ in range(2):
            t = _selu(jnp.dot(he, w1e_r[...], preferred_element_type=_F32) + c)
            t = _selu(jnp.dot(t, uw2_r[...], preferred_element_type=_F32) + ub2_r[...])
            he = jnp.dot(t, uw3_r[...], preferred_element_type=_F32) + ub3_r[...]
        out_r[...] = he

    _w = lambda shape: pl.BlockSpec(shape, lambda i: (0, 0))

    def _edge_call(trg, tsg, ec):
        return pl.pallas_call(
            _edge_k,
            grid=(e8h // BR,),
            in_specs=[
                pl.BlockSpec((BR, 128), lambda i: (i, 0)),
                pl.BlockSpec((BR, 128), lambda i: (i, 0)),
                pl.BlockSpec((8, 128), lambda i: (i, 0)),
                pl.BlockSpec((8, 128), lambda i: (i, 0)),
                pl.BlockSpec((8, 128), lambda i: (i, 0)),
                _w((8, 256)), _w((8, 256)), _w((8, 256)),
                _w((1, 128)), _w((128, 256)), _w((1, 256)), _w((256, 128)),
                _w((1, 128)), _w((128, 256)), _w((128, 256)), _w((1, 256)),
                _w((256, 256)), _w((1, 256)), _w((256, 128)), _w((1, 128)),
            ],
            out_specs=pl.BlockSpec((BR, 128), lambda i: (i, 0)),
            out_shape=jax.ShapeDtypeStruct((e8h, 128), _F32),
        )(trg, tsg, ec[0], ec[1], ec[2], qmats[0], qmats[1], qmats[2],
          b1t, w2bd, b2t, w3bd, b3t, uw1n, uw1e, ub1t, uw2bd, ub2t,
          uw3bd, ub3t)

    chs = 1024
    sub = chs // 128
    per_w = eh // 32
    nchs = per_w // chs
    rows_per_tile = npad // 16
    zeros_hbm = jnp.zeros((npad, 8), _F32)
    mesh2 = plsc.VectorSubcoreMesh(core_axis_name="c", subcore_axis_name="s")

    @functools.partial(
        pl.kernel,
        out_type=(jax.ShapeDtypeStruct((npad, 8), _F32),
                  jax.ShapeDtypeStruct((npad, 8), _F32)),
        mesh=mesh2,
        scratch_types=[
            pltpu.VMEM((sub, 128), jnp.int32),
            pltpu.VMEM((chs, 8), _F32),
            pltpu.VMEM_SHARED((npad, 8), _F32),
            pltpu.SemaphoreType.DMA,
        ],
        compiler_params=pltpu.CompilerParams(use_tc_tiling_on_sc=False),
    )
    def _scatter_k(he_hbm, ri_hbm, z_hbm, out0, out1, idx_v, val_v, acc, sem):
        c = lax.axis_index("c")
        s = lax.axis_index("s")
        wid = s * 2 + c
        tile_rows = pl.ds(s * rows_per_tile, rows_per_tile)
        pltpu.sync_copy(z_hbm.at[tile_rows], acc.at[tile_rows])
        plsc.subcore_barrier()

        def body(i):
            ebase = wid * per_w + i * chs
            pltpu.sync_copy(ri_hbm.at[pl.ds(ebase // 128, sub)], idx_v)
            pltpu.sync_copy(he_hbm.at[pl.ds(ebase, chs), pl.ds(0, 8)], val_v)
            for j in range(sub):
                pltpu.sync_copy(val_v.at[pl.ds(j * 128, 128)],
                                acc.at[idx_v.at[j]], add=True)

        pl.loop(0, nchs)(body)
        plsc.subcore_barrier()

        @pl.when(c == 0)
        def _():
            pltpu.sync_copy(acc.at[tile_rows], out0.at[tile_rows])

        @pl.when(c == 1)
        def _():
            pltpu.sync_copy(acc.at[tile_rows], out1.at[tile_rows])

    partials = []
    for h in range(2):
        sl = slice(h * eh, (h + 1) * eh)
        trg, tsg = _gather_k(tr, ts, ri_g[sl], si_g[sl])
        he_pk = _edge_call(trg.reshape(e8h, 128), tsg.reshape(e8h, 128),
                           ecols[h])
        p0, p1 = _scatter_k(he_pk.reshape(eh, 16),
                            ri_sc[h * (eh // 128):(h + 1) * (eh // 128)],
                            zeros_hbm)
        partials += [p0, p1]

    def _node_k(p0_r, p1_r, p2_r, p3_r, nd_r, pw1_r, pb1_r, pw2_r, pb2_r,
                pw3_r, pb3_r, nw1n_r, nw1e_r, nb1_r, nw2_r, nb2_r, nw3_r,
                nb3_r, out_r):
        i = pl.program_id(0)
        seg = (p0_r[...] + p1_r[...]) + (p2_r[...] + p3_r[...])
        h = _selu(jnp.dot(seg, pw1_r[...], preferred_element_type=_F32) + pb1_r[...])
        h = _selu(jnp.dot(h, pw2_r[...], preferred_element_type=_F32) + pb2_r[...])
        h2e = jnp.dot(h, pw3_r[...], preferred_element_type=_F32) + pb3_r[...]
        c2 = jnp.dot(h2e, nw1e_r[...], preferred_element_type=_F32) + nb1_r[...]
        hn = nd_r[...]
        for _ in range(3):
            t = _selu(jnp.dot(hn, nw1n_r[...], preferred_element_type=_F32) + c2)
            t = _selu(jnp.dot(t, nw2_r[...], preferred_element_type=_F32) + nb2_r[...])
            hn = jnp.dot(t, nw3_r[...], preferred_element_type=_F32) + nb3_r[...]
        g = jnp.concatenate([hn[:, :3], h2e, jnp.zeros((NBK, 5), _F32)], axis=1)
        rows = i * NBK + lax.broadcasted_iota(jnp.int32, (NBK, 1), 0)
        g = jnp.where(rows < N, g, 0.0)
        gs = jnp.sum(g, axis=0, keepdims=True)
        contrib = jnp.concatenate([gs, jnp.zeros((7, 24), _F32)], axis=0)

        @pl.when(i == 0)
        def _():
            out_r[...] = jnp.zeros((8, 24), _F32)

        out_r[...] += contrib

    gsum = pl.pallas_call(
        _node_k,
        grid=(npad // NBK,),
        in_specs=[
            pl.BlockSpec((NBK, 8), lambda i: (i, 0)),
            pl.BlockSpec((NBK, 8), lambda i: (i, 0)),
            pl.BlockSpec((NBK, 8), lambda i: (i, 0)),
            pl.BlockSpec((NBK, 8), lambda i: (i, 0)),
            pl.BlockSpec((NBK, 8), lambda i: (i, 0)),
            _w((8, 16)), _w((1, 16)), _w((16, 32)), _w((1, 32)), _w((32, 16)),
            _w((1, 16)), _w((8, 32)), _w((16, 32)), _w((1, 32)), _w((32, 16)),
            _w((1, 16)), _w((16, 8)), _w((1, 8)),
        ],
        out_specs=pl.BlockSpec((8, 24), lambda i: (0, 0)),
        out_shape=jax.ShapeDtypeStruct((8, 24), _F32),
    )(partials[0], partials[1], partials[2], partials[3], nodes_p8,
      pw1, pb1, pw2, pb2, pw3, pb3, nw1n, nw1e, nb1, nw2, nb2, nw3, nb3)

    def _pred_k(x_r, w1_r, b1_r, w2_r, b2_r, w3_r, b3_r, out_r):
        h = _selu(jnp.dot(x_r[...], w1_r[...], preferred_element_type=_F32) + b1_r[...])
        h = _selu(jnp.dot(h, w2_r[...], preferred_element_type=_F32) + b2_r[...])
        out_r[...] = jnp.dot(h, w3_r[...], preferred_element_type=_F32) + b3_r[...]

    out = pl.pallas_call(
        _pred_k,
        out_shape=jax.ShapeDtypeStruct((8, 16), _F32),
    )(gsum, qw1, qb1, qw2, qb2, qw3, qb3)

    return out[0:1, 0:9]


# speedup vs baseline: 1.0472x; 1.0129x over previous
import functools

import jax
import jax.numpy as jnp
from jax import lax
from jax.experimental import pallas as pl
from jax.experimental.pallas import tpu as pltpu
from jax.experimental.pallas import tpu_sc as plsc

_F32 = jnp.float32
_SELU_SCALE = 1.0507009873554805
_SELU_ALPHA = 1.6732632423543772


def _selu(x):
    return jnp.where(x > 0, x, _SELU_ALPHA * jnp.exp(x) - _SELU_ALPHA)


def _pad2(w, r, c):
    w = jnp.asarray(w, _F32)
    return jnp.zeros((r, c), _F32).at[: w.shape[0], : w.shape[1]].set(w)


def _row(b, c):
    b = jnp.asarray(b, _F32).reshape(1, -1)
    return _pad2(b, 1, c)


def _bdg(w, gi, go):
    return jnp.kron(jnp.eye(8, dtype=_F32), _pad2(w, gi, go))


def _tileg(b, g):
    return jnp.tile(_row(b, g), (1, 8))


def kernel(nodes, edges, senders, receivers, params):
    N, E = nodes.shape[0], edges.shape[0]
    NBK = 2048
    npad = -(-N // NBK) * NBK
    epad = -(-E // 65536) * 65536
    eh = epad // 2
    e8h = eh // 8

    senders = jnp.asarray(senders, jnp.int32)
    receivers = jnp.asarray(receivers, jnp.int32)
    pad_e = epad - E

    nblk = epad // 1024

    def _perm(x):
        return x.reshape(nblk, 8, 128).transpose(0, 2, 1).reshape(epad)

    pad_gather = jnp.arange(pad_e, dtype=jnp.int32) % N
    pad_scatter = N + (jnp.arange(pad_e, dtype=jnp.int32) % 8)
    ri_g = _perm(jnp.concatenate([receivers, pad_gather]))
    si_g = _perm(jnp.concatenate([senders, pad_gather]))
    ri_sc = _perm(jnp.concatenate([receivers, pad_scatter])).reshape(
        epad // 128, 128)

    nodes_p8 = _pad2(nodes, npad, 8)
    edt = jnp.pad(edges.astype(_F32).T, ((0, 0), (0, pad_e)))
    ecols = [[edt[j, h * eh:(h + 1) * eh].reshape(eh // 128, 128)
              for j in range(3)] for h in range(2)]

    pn, ue, pe, un, pr = (params["permutate_nodes"], params["update_edges"],
                          params["permutate_edges"], params["update_nodes"],
                          params["predict"])
    w1r = _pad2(pn[0][0][:3], 8, 16)
    w1s = _pad2(pn[0][0][3:], 8, 16)
    b1t = _tileg(pn[0][1], 16)
    _S = _SELU_SCALE
    w2bd, b2t = _bdg(_S * pn[1][0], 16, 32), _tileg(pn[1][1], 32)
    w3bd, b3t = _bdg(_S * pn[2][0], 32, 16), _tileg(pn[2][1], 16)
    uw1e = _bdg(ue[0][0][:3], 16, 32)
    qmats = [jnp.kron(jnp.eye(8, dtype=_F32), _row(ue[0][0][j], 32))
             for j in range(3)]
    uw1n = _bdg(ue[0][0][3:], 16, 32)
    ub1t = _tileg(ue[0][1], 32)
    uw2bd, ub2t = _bdg(_S * ue[1][0], 32, 32), _tileg(ue[1][1], 32)
    uw3bd, ub3t = _bdg(_S * ue[2][0], 32, 16), _tileg(ue[2][1], 16)
    pw1, pb1 = _pad2(pe[0][0], 8, 16), _row(pe[0][1], 16)
    pw2, pb2 = _pad2(_S * pe[1][0], 16, 32), _row(pe[1][1], 32)
    pw3, pb3 = _pad2(_S * pe[2][0], 32, 16), _row(pe[2][1], 16)
    nw1n = _pad2(un[0][0][:3], 8, 32)
    nw1e = _pad2(un[0][0][3:], 16, 32)
    nb1 = _row(un[0][1], 32)
    nw2, nb2 = _pad2(_S * un[1][0], 32, 16), _row(un[1][1], 16)
    nw3, nb3 = _pad2(_S * un[2][0], 16, 8), _row(un[2][1], 8)
    qw1, qb1 = _pad2(pr[0][0], 24, 16), _row(pr[0][1], 16)
    qw2, qb2 = _pad2(_S * pr[1][0], 16, 16), _row(pr[1][1], 16)
    qw3, qb3 = _pad2(_S * pr[2][0], 16, 16), _row(pr[2][1], 16)

    def _tables_k(x_ref, wr_ref, ws_ref, tr_ref, ts_ref):
        x = x_ref[...]
        tr_ref[...] = jnp.dot(x, wr_ref[...], preferred_element_type=_F32)
        ts_ref[...] = jnp.dot(x, ws_ref[...], preferred_element_type=_F32)

    tr, ts = pl.pallas_call(
        _tables_k,
        grid=(npad // NBK,),
        in_specs=[
            pl.BlockSpec((NBK, 8), lambda i: (i, 0)),
            pl.BlockSpec((8, 16), lambda i: (0, 0)),
            pl.BlockSpec((8, 16), lambda i: (0, 0)),
        ],
        out_specs=[pl.BlockSpec((NBK, 16), lambda i: (i, 0))] * 2,
        out_shape=[jax.ShapeDtypeStruct((npad, 16), _F32)] * 2,
    )(nodes_p8, w1r, w1s)

    chg = 1024
    per_tile = eh // 16
    nch = per_tile // chg
    mesh = plsc.VectorSubcoreMesh(core_axis_name="c", subcore_axis_name="s")

    @functools.partial(
        pl.kernel,
        out_type=(jax.ShapeDtypeStruct((eh, 16), _F32),
                  jax.ShapeDtypeStruct((eh, 16), _F32)),
        mesh=mesh,
        scratch_types=[
            pltpu.VMEM((chg,), jnp.int32),
            pltpu.VMEM((chg, 16), _F32),
            pltpu.SemaphoreType.DMA,
        ],
        compiler_params=pltpu.CompilerParams(use_tc_tiling_on_sc=False),
    )
    def _gather_k(tr_hbm, ts_hbm, ri_hbm, si_hbm, outr, outs, idx_v, rows_v, sem):
        c = lax.axis_index("c")
        s = lax.axis_index("s")

        def run(tab, idxs, out):
            def body(i):
                base = s * per_tile + i * chg
                pltpu.sync_copy(idxs.at[pl.ds(base, chg)], idx_v)
                pltpu.async_copy(tab.at[idx_v], rows_v, sem).wait()
                pltpu.sync_copy(rows_v, out.at[pl.ds(base, chg)])
            pl.loop(0, nch)(body)

        @pl.when(c == 0)
        def _():
            run(tr_hbm, ri_hbm, outr)

        @pl.when(c == 1)
        def _():
            run(ts_hbm, si_hbm, outs)

    BR = 128

    def _edge_k(trg_r, tsg_r, e0_r, e1_r, e2_r, q0_r, q1_r, q2_r, b1_r, w2_r,
                b2_r, w3_r, b3_r, w1n_r, w1e_r, ub1_r, uw2_r, ub2_r, uw3_r,
                ub3_r, out_r):
        h = _selu(trg_r[...] + tsg_r[...] + b1_r[...])
        h = _selu(jnp.dot(h, w2_r[...], preferred_element_type=_F32) + b2_r[...])
        hn = jnp.dot(h, w3_r[...], preferred_element_type=_F32) + b3_r[...]
        c = jnp.dot(hn, w1n_r[...], preferred_element_type=_F32) + ub1_r[...]
        dn = (((0,), (0,)), ((), ()))
        edc = (lax.dot_general(e0_r[...], q0_r[...], dn, preferred_element_type=_F32)
               + lax.dot_general(e1_r[...], q1_r[...], dn, preferred_element_type=_F32)
               + lax.dot_general(e2_r[...], q2_r[...], dn, preferred_element_type=_F32))
        t = _selu(edc + c)
        t = _selu(jnp.dot(t, uw2_r[...], preferred_element_type=_F32) + ub2_r[...])
        he = jnp.dot(t, uw3_r[...], preferred_element_type=_F32) + ub3_r[...]
        for _ in range(2):
            t = _selu(jnp.dot(he, w1e_r[...], preferred_element_type=_F32) + c)
            t = _selu(jnp.dot(t, uw2_r[...], preferred_element_type=_F32) + ub2_r[...])
            he = jnp.dot(t, uw3_r[...], preferred_element_type=_F32) + ub3_r[...]
        out_r[...] = he

    _w = lambda shape: pl.BlockSpec(shape, lambda i: (0, 0))

    def _edge_call(trg, tsg, ec):
        return pl.pallas_call(
            _edge_k,
            grid=(e8h // BR,),
            in_specs=[
                pl.BlockSpec((BR, 128), lambda i: (i, 0)),
                pl.BlockSpec((BR, 128), lambda i: (i, 0)),
                pl.BlockSpec((8, 128), lambda i: (i, 0)),
                pl.BlockSpec((8, 128), lambda i: (i, 0)),
                pl.BlockSpec((8, 128), lambda i: (i, 0)),
                _w((8, 256)), _w((8, 256)), _w((8, 256)),
                _w((1, 128)), _w((128, 256)), _w((1, 256)), _w((256, 128)),
                _w((1, 128)), _w((128, 256)), _w((128, 256)), _w((1, 256)),
                _w((256, 256)), _w((1, 256)), _w((256, 128)), _w((1, 128)),
            ],
            out_specs=pl.BlockSpec((BR, 128), lambda i: (i, 0)),
            out_shape=jax.ShapeDtypeStruct((e8h, 128), _F32),
        )(trg, tsg, ec[0], ec[1], ec[2], qmats[0], qmats[1], qmats[2],
          b1t, w2bd, b2t, w3bd, b3t, uw1n, uw1e, ub1t, uw2bd, ub2t,
          uw3bd, ub3t)

    chs = 1024
    sub = chs // 128
    per_w = eh // 32
    nchs = per_w // chs
    rows_per_tile = npad // 16
    zeros_hbm = jnp.zeros((npad, 8), _F32)
    mesh2 = plsc.VectorSubcoreMesh(core_axis_name="c", subcore_axis_name="s")

    @functools.partial(
        pl.kernel,
        out_type=(jax.ShapeDtypeStruct((npad, 8), _F32),
                  jax.ShapeDtypeStruct((npad, 8), _F32)),
        mesh=mesh2,
        scratch_types=[
            pltpu.VMEM((sub, 128), jnp.int32),
            pltpu.VMEM((chs, 8), _F32),
            pltpu.VMEM_SHARED((npad, 8), _F32),
            pltpu.SemaphoreType.DMA,
        ],
        compiler_params=pltpu.CompilerParams(use_tc_tiling_on_sc=False),
    )
    def _scatter_k(he_hbm, ri_hbm, z_hbm, out0, out1, idx_v, val_v, acc, sem):
        c = lax.axis_index("c")
        s = lax.axis_index("s")
        wid = s * 2 + c
        tile_rows = pl.ds(s * rows_per_tile, rows_per_tile)
        pltpu.sync_copy(z_hbm.at[tile_rows], acc.at[tile_rows])
        plsc.subcore_barrier()

        def body(i):
            ebase = wid * per_w + i * chs
            pltpu.sync_copy(ri_hbm.at[pl.ds(ebase // 128, sub)], idx_v)
            pltpu.sync_copy(he_hbm.at[pl.ds(ebase, chs), pl.ds(0, 8)], val_v)
            for j in range(sub):
                pltpu.sync_copy(val_v.at[pl.ds(j * 128, 128)],
                                acc.at[idx_v.at[j]], add=True)

        pl.loop(0, nchs)(body)
        plsc.subcore_barrier()

        @pl.when(c == 0)
        def _():
            pltpu.sync_copy(acc.at[tile_rows], out0.at[tile_rows])

        @pl.when(c == 1)
        def _():
            pltpu.sync_copy(acc.at[tile_rows], out1.at[tile_rows])

    partials = []
    for h in range(2):
        sl = slice(h * eh, (h + 1) * eh)
        trg, tsg = _gather_k(tr, ts, ri_g[sl], si_g[sl])
        he_pk = _edge_call(trg.reshape(e8h, 128), tsg.reshape(e8h, 128),
                           ecols[h])
        p0, p1 = _scatter_k(he_pk.reshape(eh, 16),
                            ri_sc[h * (eh // 128):(h + 1) * (eh // 128)],
                            zeros_hbm)
        partials += [p0, p1]

    def _node_k(p0_r, p1_r, p2_r, p3_r, nd_r, pw1_r, pb1_r, pw2_r, pb2_r,
                pw3_r, pb3_r, nw1n_r, nw1e_r, nb1_r, nw2_r, nb2_r, nw3_r,
                nb3_r, out_r):
        i = pl.program_id(0)
        seg = (p0_r[...] + p1_r[...]) + (p2_r[...] + p3_r[...])
        h = _selu(jnp.dot(seg, pw1_r[...], preferred_element_type=_F32) + pb1_r[...])
        h = _selu(jnp.dot(h, pw2_r[...], preferred_element_type=_F32) + pb2_r[...])
        h2e = jnp.dot(h, pw3_r[...], preferred_element_type=_F32) + pb3_r[...]
        c2 = jnp.dot(h2e, nw1e_r[...], preferred_element_type=_F32) + nb1_r[...]
        hn = nd_r[...]
        for _ in range(3):
            t = _selu(jnp.dot(hn, nw1n_r[...], preferred_element_type=_F32) + c2)
            t = _selu(jnp.dot(t, nw2_r[...], preferred_element_type=_F32) + nb2_r[...])
            hn = jnp.dot(t, nw3_r[...], preferred_element_type=_F32) + nb3_r[...]
        g = jnp.concatenate([hn[:, :3], h2e, jnp.zeros((NBK, 5), _F32)], axis=1)
        rows = i * NBK + lax.broadcasted_iota(jnp.int32, (NBK, 1), 0)
        g = jnp.where(rows < N, g, 0.0)
        gs = jnp.sum(g, axis=0, keepdims=True)
        contrib = jnp.concatenate([gs, jnp.zeros((7, 24), _F32)], axis=0)

        @pl.when(i == 0)
        def _():
            out_r[...] = jnp.zeros((8, 24), _F32)

        out_r[...] += contrib

    gsum = pl.pallas_call(
        _node_k,
        grid=(npad // NBK,),
        in_specs=[
            pl.BlockSpec((NBK, 8), lambda i: (i, 0)),
            pl.BlockSpec((NBK, 8), lambda i: (i, 0)),
            pl.BlockSpec((NBK, 8), lambda i: (i, 0)),
            pl.BlockSpec((NBK, 8), lambda i: (i, 0)),
            pl.BlockSpec((NBK, 8), lambda i: (i, 0)),
            _w((8, 16)), _w((1, 16)), _w((16, 32)), _w((1, 32)), _w((32, 16)),
            _w((1, 16)), _w((8, 32)), _w((16, 32)), _w((1, 32)), _w((32, 16)),
            _w((1, 16)), _w((16, 8)), _w((1, 8)),
        ],
        out_specs=pl.BlockSpec((8, 24), lambda i: (0, 0)),
        out_shape=jax.ShapeDtypeStruct((8, 24), _F32),
    )(partials[0], partials[1], partials[2], partials[3], nodes_p8,
      pw1, pb1, pw2, pb2, pw3, pb3, nw1n, nw1e, nb1, nw2, nb2, nw3, nb3)

    def _pred_k(x_r, w1_r, b1_r, w2_r, b2_r, w3_r, b3_r, out_r):
        h = _selu(jnp.dot(x_r[...], w1_r[...], preferred_element_type=_F32) + b1_r[...])
        h = _selu(jnp.dot(h, w2_r[...], preferred_element_type=_F32) + b2_r[...])
        out_r[...] = jnp.dot(h, w3_r[...], preferred_element_type=_F32) + b3_r[...]

    out = pl.pallas_call(
        _pred_k,
        out_shape=jax.ShapeDtypeStruct((8, 16), _F32),
    )(gsum, qw1, qb1, qw2, qb2, qw3, qb3)

    return out[0:1, 0:9]
